# baseline (speedup 1.0000x reference)
import functools

import jax
import jax.numpy as jnp
from jax import lax
from jax.experimental import pallas as pl
from jax.experimental.pallas import tpu as pltpu
from jax.experimental.pallas import tpu_sc as plsc

N = 10000
F_IN = 128
EMB = 16
NCLS = 16
R_RAW = 2
NR = 2 * R_RAW + 1
E_RAW = 320000
TBL = N * NR

NC = 2
NS = 16
NT = NC * NS
LANES = 16

BATCH = 128
KB0 = 160
KB1 = 160
EP = (KB0 + KB1) * NS * BATCH

NP0, NP1 = 313, 312
SZ0, SZ1 = 5 * NP0, 5 * NP1
SZP = 1568
KC = SZP // LANES
CLOAD = 1576
SZC = 1584
KCC = SZC // LANES
ROWS_PER_SUB = TBL // NS
TBL_P = 50176
CNT_PER_SUB = TBL_P // NS
PAD_ROW = NR - 1

_mesh = plsc.VectorSubcoreMesh(
    core_axis_name="c", subcore_axis_name="s", num_cores=NC, num_subcores=NS
)


NBUF = 8
CH = 32
EPT = KB0 * BATCH


@functools.partial(
    pl.kernel,
    out_type=jax.ShapeDtypeStruct((NT, TBL_P), jnp.float32),
    mesh=_mesh,
    scratch_types=[
        pltpu.VMEM((EPT,), jnp.int32),
        pltpu.VMEM((TBL_P,), jnp.float32),
    ],
    compiler_params=pltpu.CompilerParams(
        use_tc_tiling_on_sc=False, needs_layout_passes=False
    ),
)
def _count_kernel(tidx_hbm, zeros_hbm, out_hbm, idx_v, hist_v):
    c = lax.axis_index("c")
    s = lax.axis_index("s")
    w = c * NS + s
    pltpu.sync_copy(tidx_hbm.at[pl.ds(w * EPT, EPT)], idx_v)
    pltpu.sync_copy(zeros_hbm, hist_v)
    ones = jnp.full((LANES,), 1.0, jnp.float32)

    @pl.loop(0, EPT // LANES)
    def _(i):
        v = idx_v[pl.ds(i * LANES, LANES)]
        plsc.addupdate_scatter(hist_v, [v], ones)

    pltpu.sync_copy(hist_v, out_hbm.at[w])


@functools.partial(
    pl.kernel,
    out_type=jax.ShapeDtypeStruct((NC, NS, ROWS_PER_SUB, EMB), jnp.float32),
    mesh=_mesh,
    scratch_types=(
        [pltpu.VMEM((CH, BATCH), jnp.int32) for _ in range(4)]
        + [pltpu.VMEM((BATCH, EMB), jnp.float32) for _ in range(NBUF)]
        + [pltpu.VMEM_SHARED((TBL, EMB), jnp.float32)]
        + [pltpu.SemaphoreType.DMA for _ in range(NBUF + 2)]
    ),
    compiler_params=pltpu.CompilerParams(
        use_tc_tiling_on_sc=False, needs_layout_passes=False
    ),
)
def _edge_pass(y_hbm, g_hbm, t_hbm, z_hbm, acc_out, *refs):
    gi = refs[0:2]
    ti = refs[2:4]
    bufs = refs[4:4 + NBUF]
    acc_sh = refs[4 + NBUF]
    sems = refs[5 + NBUF:5 + 2 * NBUF]
    isems = refs[5 + 2 * NBUF:7 + 2 * NBUF]

    c = lax.axis_index("c")
    s = lax.axis_index("s")
    pltpu.sync_copy(z_hbm, acc_sh.at[pl.ds(s * ROWS_PER_SUB, ROWS_PER_SUB)])
    plsc.subcore_barrier()

    def pipeline(base, kb):
        nchk = kb // CH
        pltpu.async_copy(g_hbm.at[pl.ds(base, CH)], gi[0], isems[0])
        pltpu.async_copy(t_hbm.at[pl.ds(base, CH)], ti[0], isems[0])
        for ch in range(nchk):
            p = ch % 2
            if ch + 1 < nchk:
                off = base + (ch + 1) * CH
                pltpu.async_copy(g_hbm.at[pl.ds(off, CH)], gi[1 - p],
                                 isems[1 - p])
                pltpu.async_copy(t_hbm.at[pl.ds(off, CH)], ti[1 - p],
                                 isems[1 - p])
            off = base + ch * CH
            pltpu.make_async_copy(
                g_hbm.at[pl.ds(off, CH)], gi[p], isems[p]).wait()
            pltpu.make_async_copy(
                t_hbm.at[pl.ds(off, CH)], ti[p], isems[p]).wait()

            for b in range(NBUF):
                pltpu.async_copy(y_hbm.at[gi[p].at[b]], bufs[b], sems[b])

            @pl.loop(0, CH, step=NBUF)
            def _(j):
                for b in range(NBUF):
                    jj = j + b
                    pltpu.make_async_copy(
                        y_hbm.at[gi[p].at[jj]], bufs[b], sems[b]).wait()
                    pltpu.sync_copy(bufs[b], acc_sh.at[ti[p].at[jj]],
                                    add=True)

                    @pl.when(jj + NBUF < CH)
                    def _():
                        pltpu.async_copy(
                            y_hbm.at[gi[p].at[jj + NBUF]], bufs[b], sems[b])

    @pl.when(c == 0)
    def _():
        pipeline(s * KB0, KB0)

    @pl.when(c == 1)
    def _():
        pipeline(NS * KB0 + s * KB1, KB1)

    plsc.subcore_barrier()
    sl = pl.ds(s * ROWS_PER_SUB, ROWS_PER_SUB)
    pltpu.sync_copy(acc_sh.at[sl], acc_out.at[c, s])


def _splat(vec, lane):
    return lax.gather(
        vec, jnp.full((LANES, 1), lane, jnp.int32),
        lax.GatherDimensionNumbers(offset_dims=(), collapsed_slice_dims=(0,),
                                   start_index_map=(0,)),
        (1,), mode=lax.GatherScatterMode.PROMISE_IN_BOUNDS)


def _make_finish(relu):
    scratch = [
        pltpu.VMEM((SZP, EMB), jnp.float32),
        pltpu.VMEM((SZP, EMB), jnp.float32),
        pltpu.VMEM((SZP, EMB), jnp.float32),
        pltpu.VMEM((8, SZC), jnp.float32),
        pltpu.VMEM((SZC,), jnp.float32),
        pltpu.VMEM((SZC, EMB), jnp.float32),
        pltpu.VMEM((320, EMB), jnp.float32),
        pltpu.VMEM((LANES,), jnp.float32),
        pltpu.SemaphoreType.DMA,
    ]

    def body(acc_hbm, cnt_hbm, y_hbm, b_hbm, z1_hbm, out_hbm,
             a0_v, a1_v, y_v, cb_v, cs_v, inv_v, h_v, b_v, sem):
        c = lax.axis_index("c")
        s = lax.axis_index("s")
        pltpu.sync_copy(b_hbm, b_v)

        def run(half, sz, np_, nbase):
            row_lo = s * ROWS_PER_SUB + half
            pltpu.sync_copy(acc_hbm.at[0, s].at[pl.ds(half, sz)],
                            a0_v.at[pl.ds(0, sz)])
            pltpu.sync_copy(acc_hbm.at[1, s].at[pl.ds(half, sz)],
                            a1_v.at[pl.ds(0, sz)])
            pltpu.sync_copy(y_hbm.at[pl.ds(row_lo, sz)], y_v.at[pl.ds(0, sz)])
            clo = pl.multiple_of(row_lo & ~7, 8)
            dc = row_lo - clo
            pltpu.sync_copy(z1_hbm.at[pl.ds(0, SZC)], cs_v)
            for u in range(8):
                pltpu.sync_copy(z1_hbm.at[pl.ds(0, SZC)], cb_v.at[u])
            for g in range(NT // 8):
                for u in range(8):
                    pltpu.async_copy(
                        cnt_hbm.at[g * 8 + u].at[pl.ds(clo, CLOAD)],
                        cb_v.at[u].at[pl.ds(0, CLOAD)], sem)
                for u in range(8):
                    pltpu.make_async_copy(
                        cnt_hbm.at[g * 8 + u].at[pl.ds(clo, CLOAD)],
                        cb_v.at[u].at[pl.ds(0, CLOAD)], sem).wait()

                @pl.loop(0, KCC)
                def _(k):
                    sl = pl.ds(k * LANES, LANES)
                    acc = cs_v[sl]
                    for u in range(8):
                        acc = acc + cb_v[u, sl]
                    cs_v[sl] = acc

            @pl.loop(0, KCC)
            def _(k):
                sl = pl.ds(k * LANES, LANES)
                cv = cs_v[sl]
                iv = jnp.where(cv > 0.0, 1.0 / cv, 0.0)
                for l in range(LANES):
                    inv_v[k * LANES + l] = _splat(iv, l)

            bias = b_v[...]

            @pl.loop(0, np_)
            def _(i):
                j = 5 * i
                msg = y_v[j + NR - 1] + bias
                for r in range(NR - 1):
                    msg = msg + (a0_v[j + r] + a1_v[j + r]) * inv_v[dc + j + r]
                if relu:
                    msg = jnp.maximum(msg, 0.0)
                h_v[i] = msg

            pltpu.sync_copy(h_v.at[pl.ds(0, np_)],
                            out_hbm.at[pl.ds(nbase, np_)])

        @pl.when(c == 0)
        def _():
            run(0, SZ0, NP0, s * 625)

        @pl.when(c == 1)
        def _():
            run(SZ0, SZ1, NP1, s * 625 + NP0)

    return pl.kernel(
        body,
        out_type=jax.ShapeDtypeStruct((N, EMB), jnp.float32),
        mesh=_mesh,
        scratch_types=scratch,
        compiler_params=pltpu.CompilerParams(
            use_tc_tiling_on_sc=False, needs_layout_passes=False
        ),
    )


_finish_relu = _make_finish(True)
_finish_plain = _make_finish(False)


def _dense_body(x_ref, w_ref, y_ref):
    x = x_ref[...]
    for r in range(NR):
        y_ref[:, r, :] = jnp.dot(x, w_ref[r], preferred_element_type=jnp.float32)


def _dense_call(x, w, k_in):
    nb = 1000
    y = pl.pallas_call(
        _dense_body,
        grid=(N // nb,),
        in_specs=[
            pl.BlockSpec((nb, k_in), lambda i: (i, 0)),
            pl.BlockSpec((NR, k_in, EMB), lambda i: (0, 0, 0)),
        ],
        out_specs=pl.BlockSpec((nb, NR, EMB), lambda i: (i, 0, 0)),
        out_shape=jax.ShapeDtypeStruct((N, NR, EMB), jnp.float32),
    )(x, w)
    return y.reshape(TBL, EMB)


def kernel(features, src, rel, dst, w1, bias1, w2, bias2):
    f32 = jnp.float32
    pad = EP - 2 * E_RAW
    gidx = jnp.concatenate([
        dst * NR + rel, src * NR + (rel + R_RAW),
        jnp.full((pad,), PAD_ROW, jnp.int32),
    ])
    tidx = jnp.concatenate([
        src * NR + rel, dst * NR + (rel + R_RAW),
        jnp.full((pad,), PAD_ROW, jnp.int32),
    ])
    gidx2 = gidx.reshape(EP // BATCH, BATCH)
    tidx2 = tidx.reshape(EP // BATCH, BATCH)
    zeros2 = jnp.zeros((ROWS_PER_SUB, EMB), f32)
    zeros1 = jnp.zeros((TBL_P,), f32)

    counts = _count_kernel(tidx, zeros1)
    y1 = _dense_call(features, w1, F_IN)
    acc1 = _edge_pass(y1, gidx2, tidx2, zeros2)
    h = _finish_relu(acc1, counts, y1, bias1, zeros1)
    y2 = _dense_call(h, w2, EMB)
    acc2 = _edge_pass(y2, gidx2, tidx2, zeros2)
    return _finish_plain(acc2, counts, y2, bias2, zeros1)


# speedup vs baseline: 1.1070x; 1.1070x over previous
import functools

import jax
import jax.numpy as jnp
from jax import lax
from jax.experimental import pallas as pl
from jax.experimental.pallas import tpu as pltpu
from jax.experimental.pallas import tpu_sc as plsc

N = 10000
F_IN = 128
EMB = 16
NCLS = 16
R_RAW = 2
NR = 2 * R_RAW + 1
E_RAW = 320000
TBL = N * NR

NC = 2
NS = 16
NT = NC * NS
LANES = 16

BATCH = 128
KB0 = 160
KB1 = 160
EP = (KB0 + KB1) * NS * BATCH

NP0, NP1 = 313, 312
SZ0, SZ1 = 5 * NP0, 5 * NP1
SZP = 1568
KC = SZP // LANES
CLOAD = 1576
SZC = 1584
KCC = SZC // LANES
ROWS_PER_SUB = TBL // NS
TBL_P = 50176
CNT_PER_SUB = TBL_P // NS
PAD_ROW = NR - 1

_mesh = plsc.VectorSubcoreMesh(
    core_axis_name="c", subcore_axis_name="s", num_cores=NC, num_subcores=NS
)


NBUF = 4
CH = 32


def _make_edge_pass(with_counts):
    out_types = [jax.ShapeDtypeStruct((NC, NS, ROWS_PER_SUB, EMB), jnp.float32)]
    scratch = (
        [pltpu.VMEM((CH, BATCH), jnp.int32) for _ in range(4)]
        + [pltpu.VMEM((BATCH, EMB), jnp.float32) for _ in range(NBUF)]
        + [pltpu.VMEM_SHARED((TBL, EMB), jnp.float32)]
        + [pltpu.SemaphoreType.DMA for _ in range(NBUF + 2)]
    )
    if with_counts:
        out_types.append(jax.ShapeDtypeStruct((NT, TBL_P), jnp.float32))
        scratch.append(pltpu.VMEM((TBL_P,), jnp.float32))

    def body(y_hbm, g_hbm, t_hbm, z_hbm, z1_hbm, *refs):
        if with_counts:
            acc_out, cnt_out = refs[0], refs[1]
            refs = refs[2:]
        else:
            acc_out = refs[0]
            refs = refs[1:]
        gi = refs[0:2]
        ti = refs[2:4]
        bufs = refs[4:4 + NBUF]
        acc_sh = refs[4 + NBUF]
        sems = refs[5 + NBUF:5 + 2 * NBUF]
        isems = refs[5 + 2 * NBUF:7 + 2 * NBUF]
        hist_v = refs[7 + 2 * NBUF] if with_counts else None

        c = lax.axis_index("c")
        s = lax.axis_index("s")
        w = c * NS + s
        pltpu.sync_copy(z_hbm, acc_sh.at[pl.ds(s * ROWS_PER_SUB, ROWS_PER_SUB)])
        if with_counts:
            pltpu.sync_copy(z1_hbm, hist_v)
        plsc.subcore_barrier()

        ones = jnp.full((LANES,), 1.0, jnp.float32)

        def pipeline(base, kb):
            nchk = kb // CH
            pltpu.async_copy(g_hbm.at[pl.ds(base, CH)], gi[0], isems[0])
            pltpu.async_copy(t_hbm.at[pl.ds(base, CH)], ti[0], isems[0])
            for ch in range(nchk):
                p = ch % 2
                if ch + 1 < nchk:
                    off = base + (ch + 1) * CH
                    pltpu.async_copy(g_hbm.at[pl.ds(off, CH)], gi[1 - p],
                                     isems[1 - p])
                    pltpu.async_copy(t_hbm.at[pl.ds(off, CH)], ti[1 - p],
                                     isems[1 - p])
                off = base + ch * CH
                pltpu.make_async_copy(
                    g_hbm.at[pl.ds(off, CH)], gi[p], isems[p]).wait()
                pltpu.make_async_copy(
                    t_hbm.at[pl.ds(off, CH)], ti[p], isems[p]).wait()

                for b in range(NBUF):
                    pltpu.async_copy(y_hbm.at[gi[p].at[b]], bufs[b], sems[b])

                @pl.loop(0, CH, step=NBUF)
                def _(j):
                    for b in range(NBUF):
                        jj = j + b
                        pltpu.make_async_copy(
                            y_hbm.at[gi[p].at[jj]], bufs[b], sems[b]).wait()
                        pltpu.sync_copy(bufs[b], acc_sh.at[ti[p].at[jj]],
                                        add=True)

                        @pl.when(jj + NBUF < CH)
                        def _():
                            pltpu.async_copy(
                                y_hbm.at[gi[p].at[jj + NBUF]], bufs[b], sems[b])

                        if with_counts:
                            for k in range(BATCH // LANES):
                                v = ti[p][jj, pl.ds(k * LANES, LANES)]
                                plsc.addupdate_scatter(hist_v, [v], ones)

        @pl.when(c == 0)
        def _():
            pipeline(s * KB0, KB0)

        @pl.when(c == 1)
        def _():
            pipeline(NS * KB0 + s * KB1, KB1)

        plsc.subcore_barrier()
        sl = pl.ds(s * ROWS_PER_SUB, ROWS_PER_SUB)
        pltpu.sync_copy(acc_sh.at[sl], acc_out.at[c, s])
        if with_counts:
            pltpu.sync_copy(hist_v, cnt_out.at[w])

    return pl.kernel(
        body,
        out_type=tuple(out_types) if with_counts else out_types[0],
        mesh=_mesh,
        scratch_types=scratch,
        compiler_params=pltpu.CompilerParams(
            use_tc_tiling_on_sc=False, needs_layout_passes=False
        ),
    )


_edge_pass_count = _make_edge_pass(True)
_edge_pass_plain = _make_edge_pass(False)


def _splat(vec, lane):
    return lax.gather(
        vec, jnp.full((LANES, 1), lane, jnp.int32),
        lax.GatherDimensionNumbers(offset_dims=(), collapsed_slice_dims=(0,),
                                   start_index_map=(0,)),
        (1,), mode=lax.GatherScatterMode.PROMISE_IN_BOUNDS)


def _make_finish(relu):
    scratch = [
        pltpu.VMEM((SZP, EMB), jnp.float32),
        pltpu.VMEM((SZP, EMB), jnp.float32),
        pltpu.VMEM((SZP, EMB), jnp.float32),
        pltpu.VMEM((8, SZC), jnp.float32),
        pltpu.VMEM((SZC,), jnp.float32),
        pltpu.VMEM((SZC, EMB), jnp.float32),
        pltpu.VMEM((320, EMB), jnp.float32),
        pltpu.VMEM((LANES,), jnp.float32),
        pltpu.SemaphoreType.DMA,
    ]

    def body(acc_hbm, cnt_hbm, y_hbm, b_hbm, z1_hbm, out_hbm,
             a0_v, a1_v, y_v, cb_v, cs_v, inv_v, h_v, b_v, sem):
        c = lax.axis_index("c")
        s = lax.axis_index("s")
        pltpu.sync_copy(b_hbm, b_v)

        def run(half, sz, np_, nbase):
            row_lo = s * ROWS_PER_SUB + half
            pltpu.sync_copy(acc_hbm.at[0, s].at[pl.ds(half, sz)],
                            a0_v.at[pl.ds(0, sz)])
            pltpu.sync_copy(acc_hbm.at[1, s].at[pl.ds(half, sz)],
                            a1_v.at[pl.ds(0, sz)])
            pltpu.sync_copy(y_hbm.at[pl.ds(row_lo, sz)], y_v.at[pl.ds(0, sz)])
            clo = pl.multiple_of(row_lo & ~7, 8)
            dc = row_lo - clo
            pltpu.sync_copy(z1_hbm.at[pl.ds(0, SZC)], cs_v)
            for u in range(8):
                pltpu.sync_copy(z1_hbm.at[pl.ds(0, SZC)], cb_v.at[u])
            for g in range(NT // 8):
                for u in range(8):
                    pltpu.async_copy(
                        cnt_hbm.at[g * 8 + u].at[pl.ds(clo, CLOAD)],
                        cb_v.at[u].at[pl.ds(0, CLOAD)], sem)
                for u in range(8):
                    pltpu.make_async_copy(
                        cnt_hbm.at[g * 8 + u].at[pl.ds(clo, CLOAD)],
                        cb_v.at[u].at[pl.ds(0, CLOAD)], sem).wait()

                @pl.loop(0, KCC)
                def _(k):
                    sl = pl.ds(k * LANES, LANES)
                    acc = cs_v[sl]
                    for u in range(8):
                        acc = acc + cb_v[u, sl]
                    cs_v[sl] = acc

            @pl.loop(0, KCC)
            def _(k):
                sl = pl.ds(k * LANES, LANES)
                cv = cs_v[sl]
                iv = jnp.where(cv > 0.0, 1.0 / cv, 0.0)
                for l in range(LANES):
                    inv_v[k * LANES + l] = _splat(iv, l)

            bias = b_v[...]

            @pl.loop(0, np_)
            def _(i):
                j = 5 * i
                msg = y_v[j + NR - 1] + bias
                for r in range(NR - 1):
                    msg = msg + (a0_v[j + r] + a1_v[j + r]) * inv_v[dc + j + r]
                if relu:
                    msg = jnp.maximum(msg, 0.0)
                h_v[i] = msg

            pltpu.sync_copy(h_v.at[pl.ds(0, np_)],
                            out_hbm.at[pl.ds(nbase, np_)])

        @pl.when(c == 0)
        def _():
            run(0, SZ0, NP0, s * 625)

        @pl.when(c == 1)
        def _():
            run(SZ0, SZ1, NP1, s * 625 + NP0)

    return pl.kernel(
        body,
        out_type=jax.ShapeDtypeStruct((N, EMB), jnp.float32),
        mesh=_mesh,
        scratch_types=scratch,
        compiler_params=pltpu.CompilerParams(
            use_tc_tiling_on_sc=False, needs_layout_passes=False
        ),
    )


_finish_relu = _make_finish(True)
_finish_plain = _make_finish(False)


def _dense_body(x_ref, w_ref, y_ref):
    x = x_ref[...]
    for r in range(NR):
        y_ref[:, r, :] = jnp.dot(x, w_ref[r], preferred_element_type=jnp.float32)


def _dense_call(x, w, k_in):
    nb = 1000
    y = pl.pallas_call(
        _dense_body,
        grid=(N // nb,),
        in_specs=[
            pl.BlockSpec((nb, k_in), lambda i: (i, 0)),
            pl.BlockSpec((NR, k_in, EMB), lambda i: (0, 0, 0)),
        ],
        out_specs=pl.BlockSpec((nb, NR, EMB), lambda i: (i, 0, 0)),
        out_shape=jax.ShapeDtypeStruct((N, NR, EMB), jnp.float32),
    )(x, w)
    return y.reshape(TBL, EMB)


def kernel(features, src, rel, dst, w1, bias1, w2, bias2):
    f32 = jnp.float32
    pad = EP - 2 * E_RAW
    gidx = jnp.concatenate([
        dst * NR + rel, src * NR + (rel + R_RAW),
        jnp.full((pad,), PAD_ROW, jnp.int32),
    ])
    tidx = jnp.concatenate([
        src * NR + rel, dst * NR + (rel + R_RAW),
        jnp.full((pad,), PAD_ROW, jnp.int32),
    ])
    gidx2 = gidx.reshape(EP // BATCH, BATCH)
    tidx2 = tidx.reshape(EP // BATCH, BATCH)
    zeros2 = jnp.zeros((ROWS_PER_SUB, EMB), f32)
    zeros1 = jnp.zeros((TBL_P,), f32)

    y1 = _dense_call(features, w1, F_IN)
    acc1, counts = _edge_pass_count(y1, gidx2, tidx2, zeros2, zeros1)
    h = _finish_relu(acc1, counts, y1, bias1, zeros1)
    y2 = _dense_call(h, w2, EMB)
    acc2 = _edge_pass_plain(y2, gidx2, tidx2, zeros2, zeros1)
    return _finish_plain(acc2, counts, y2, bias2, zeros1)


# speedup vs baseline: 1.1338x; 1.0242x over previous
import functools

import jax
import jax.numpy as jnp
from jax import lax
from jax.experimental import pallas as pl
from jax.experimental.pallas import tpu as pltpu
from jax.experimental.pallas import tpu_sc as plsc

N = 10000
F_IN = 128
EMB = 16
NCLS = 16
R_RAW = 2
NR = 2 * R_RAW + 1
E_RAW = 320000
TBL = N * NR

NC = 2
NS = 16
NT = NC * NS
LANES = 16

BATCH = 128
KB0 = 192
KB1 = 128
EP = (KB0 + KB1) * NS * BATCH

NP0, NP1 = 313, 312
SZ0, SZ1 = 5 * NP0, 5 * NP1
SZP = 1568
KC = SZP // LANES
CLOAD = 1576
SZC = 1584
KCC = SZC // LANES
ROWS_PER_SUB = TBL // NS
TBL_P = 50176
CNT_PER_SUB = TBL_P // NS
PAD_ROW = NR - 1

_mesh = plsc.VectorSubcoreMesh(
    core_axis_name="c", subcore_axis_name="s", num_cores=NC, num_subcores=NS
)


NBUF = 4
CH = 32


def _make_edge_pass(with_counts):
    out_types = [jax.ShapeDtypeStruct((NC, NS, ROWS_PER_SUB, EMB), jnp.float32)]
    scratch = (
        [pltpu.VMEM((CH, BATCH), jnp.int32) for _ in range(4)]
        + [pltpu.VMEM((BATCH, EMB), jnp.float32) for _ in range(NBUF)]
        + [pltpu.VMEM_SHARED((TBL, EMB), jnp.float32)]
        + [pltpu.SemaphoreType.DMA for _ in range(NBUF + 2)]
    )
    if with_counts:
        out_types.append(jax.ShapeDtypeStruct((NT, TBL_P), jnp.float32))
        scratch.append(pltpu.VMEM((TBL_P,), jnp.float32))

    def body(y_hbm, g_hbm, t_hbm, z_hbm, z1_hbm, *refs):
        if with_counts:
            acc_out, cnt_out = refs[0], refs[1]
            refs = refs[2:]
        else:
            acc_out = refs[0]
            refs = refs[1:]
        gi = refs[0:2]
        ti = refs[2:4]
        bufs = refs[4:4 + NBUF]
        acc_sh = refs[4 + NBUF]
        sems = refs[5 + NBUF:5 + 2 * NBUF]
        isems = refs[5 + 2 * NBUF:7 + 2 * NBUF]
        hist_v = refs[7 + 2 * NBUF] if with_counts else None

        c = lax.axis_index("c")
        s = lax.axis_index("s")
        w = c * NS + s
        pltpu.sync_copy(z_hbm, acc_sh.at[pl.ds(s * ROWS_PER_SUB, ROWS_PER_SUB)])
        if with_counts:
            pltpu.sync_copy(z1_hbm, hist_v)
        plsc.subcore_barrier()

        ones = jnp.full((LANES,), 1.0, jnp.float32)

        def pipeline(base, kb):
            nchk = kb // CH
            pltpu.async_copy(g_hbm.at[pl.ds(base, CH)], gi[0], isems[0])
            pltpu.async_copy(t_hbm.at[pl.ds(base, CH)], ti[0], isems[0])
            for ch in range(nchk):
                p = ch % 2
                if ch + 1 < nchk:
                    off = base + (ch + 1) * CH
                    pltpu.async_copy(g_hbm.at[pl.ds(off, CH)], gi[1 - p],
                                     isems[1 - p])
                    pltpu.async_copy(t_hbm.at[pl.ds(off, CH)], ti[1 - p],
                                     isems[1 - p])
                off = base + ch * CH
                pltpu.make_async_copy(
                    g_hbm.at[pl.ds(off, CH)], gi[p], isems[p]).wait()
                pltpu.make_async_copy(
                    t_hbm.at[pl.ds(off, CH)], ti[p], isems[p]).wait()

                for b in range(NBUF):
                    pltpu.async_copy(y_hbm.at[gi[p].at[b]], bufs[b], sems[b])

                @pl.loop(0, CH, step=NBUF)
                def _(j):
                    for b in range(NBUF):
                        jj = j + b
                        pltpu.make_async_copy(
                            y_hbm.at[gi[p].at[jj]], bufs[b], sems[b]).wait()
                        pltpu.sync_copy(bufs[b], acc_sh.at[ti[p].at[jj]],
                                        add=True)

                        @pl.when(jj + NBUF < CH)
                        def _():
                            pltpu.async_copy(
                                y_hbm.at[gi[p].at[jj + NBUF]], bufs[b], sems[b])

                        if with_counts:
                            for k in range(BATCH // LANES):
                                v = ti[p][jj, pl.ds(k * LANES, LANES)]
                                plsc.addupdate_scatter(hist_v, [v], ones)

        @pl.when(c == 0)
        def _():
            pipeline(s * KB0, KB0)

        @pl.when(c == 1)
        def _():
            pipeline(NS * KB0 + s * KB1, KB1)

        plsc.subcore_barrier()
        sl = pl.ds(s * ROWS_PER_SUB, ROWS_PER_SUB)
        pltpu.sync_copy(acc_sh.at[sl], acc_out.at[c, s])
        if with_counts:
            pltpu.sync_copy(hist_v, cnt_out.at[w])

    return pl.kernel(
        body,
        out_type=tuple(out_types) if with_counts else out_types[0],
        mesh=_mesh,
        scratch_types=scratch,
        compiler_params=pltpu.CompilerParams(
            use_tc_tiling_on_sc=False, needs_layout_passes=False
        ),
    )


_edge_pass_count = _make_edge_pass(True)
_edge_pass_plain = _make_edge_pass(False)


def _splat(vec, lane):
    return lax.gather(
        vec, jnp.full((LANES, 1), lane, jnp.int32),
        lax.GatherDimensionNumbers(offset_dims=(), collapsed_slice_dims=(0,),
                                   start_index_map=(0,)),
        (1,), mode=lax.GatherScatterMode.PROMISE_IN_BOUNDS)


def _make_finish(relu):
    scratch = [
        pltpu.VMEM((SZP, EMB), jnp.float32),
        pltpu.VMEM((SZP, EMB), jnp.float32),
        pltpu.VMEM((SZP, EMB), jnp.float32),
        pltpu.VMEM((8, SZC), jnp.float32),
        pltpu.VMEM((SZC,), jnp.float32),
        pltpu.VMEM((SZC, EMB), jnp.float32),
        pltpu.VMEM((320, EMB), jnp.float32),
        pltpu.VMEM((LANES,), jnp.float32),
        pltpu.SemaphoreType.DMA,
    ]

    def body(acc_hbm, cnt_hbm, y_hbm, b_hbm, z1_hbm, out_hbm,
             a0_v, a1_v, y_v, cb_v, cs_v, inv_v, h_v, b_v, sem):
        c = lax.axis_index("c")
        s = lax.axis_index("s")
        pltpu.sync_copy(b_hbm, b_v)

        def run(half, sz, np_, nbase):
            row_lo = s * ROWS_PER_SUB + half
            pltpu.sync_copy(acc_hbm.at[0, s].at[pl.ds(half, sz)],
                            a0_v.at[pl.ds(0, sz)])
            pltpu.sync_copy(acc_hbm.at[1, s].at[pl.ds(half, sz)],
                            a1_v.at[pl.ds(0, sz)])
            pltpu.sync_copy(y_hbm.at[pl.ds(row_lo, sz)], y_v.at[pl.ds(0, sz)])
            clo = pl.multiple_of(row_lo & ~7, 8)
            dc = row_lo - clo
            pltpu.sync_copy(z1_hbm.at[pl.ds(0, SZC)], cs_v)
            for u in range(8):
                pltpu.sync_copy(z1_hbm.at[pl.ds(0, SZC)], cb_v.at[u])
            for g in range(NT // 8):
                for u in range(8):
                    pltpu.async_copy(
                        cnt_hbm.at[g * 8 + u].at[pl.ds(clo, CLOAD)],
                        cb_v.at[u].at[pl.ds(0, CLOAD)], sem)
                for u in range(8):
                    pltpu.make_async_copy(
                        cnt_hbm.at[g * 8 + u].at[pl.ds(clo, CLOAD)],
                        cb_v.at[u].at[pl.ds(0, CLOAD)], sem).wait()

                @pl.loop(0, KCC)
                def _(k):
                    sl = pl.ds(k * LANES, LANES)
                    acc = cs_v[sl]
                    for u in range(8):
                        acc = acc + cb_v[u, sl]
                    cs_v[sl] = acc

            @pl.loop(0, KCC)
            def _(k):
                sl = pl.ds(k * LANES, LANES)
                cv = cs_v[sl]
                iv = jnp.where(cv > 0.0, 1.0 / cv, 0.0)
                for l in range(LANES):
                    inv_v[k * LANES + l] = _splat(iv, l)

            bias = b_v[...]

            @pl.loop(0, np_)
            def _(i):
                j = 5 * i
                msg = y_v[j + NR - 1] + bias
                for r in range(NR - 1):
                    msg = msg + (a0_v[j + r] + a1_v[j + r]) * inv_v[dc + j + r]
                if relu:
                    msg = jnp.maximum(msg, 0.0)
                h_v[i] = msg

            pltpu.sync_copy(h_v.at[pl.ds(0, np_)],
                            out_hbm.at[pl.ds(nbase, np_)])

        @pl.when(c == 0)
        def _():
            run(0, SZ0, NP0, s * 625)

        @pl.when(c == 1)
        def _():
            run(SZ0, SZ1, NP1, s * 625 + NP0)

    return pl.kernel(
        body,
        out_type=jax.ShapeDtypeStruct((N, EMB), jnp.float32),
        mesh=_mesh,
        scratch_types=scratch,
        compiler_params=pltpu.CompilerParams(
            use_tc_tiling_on_sc=False, needs_layout_passes=False
        ),
    )


_finish_relu = _make_finish(True)
_finish_plain = _make_finish(False)


def _dense_body(x_ref, w_ref, y_ref):
    x = x_ref[...]
    for r in range(NR):
        y_ref[:, r, :] = jnp.dot(x, w_ref[r], preferred_element_type=jnp.float32)


def _dense_call(x, w, k_in):
    nb = 1000
    y = pl.pallas_call(
        _dense_body,
        grid=(N // nb,),
        in_specs=[
            pl.BlockSpec((nb, k_in), lambda i: (i, 0)),
            pl.BlockSpec((NR, k_in, EMB), lambda i: (0, 0, 0)),
        ],
        out_specs=pl.BlockSpec((nb, NR, EMB), lambda i: (i, 0, 0)),
        out_shape=jax.ShapeDtypeStruct((N, NR, EMB), jnp.float32),
    )(x, w)
    return y.reshape(TBL, EMB)


def kernel(features, src, rel, dst, w1, bias1, w2, bias2):
    f32 = jnp.float32
    pad = EP - 2 * E_RAW
    gidx = jnp.concatenate([
        dst * NR + rel, src * NR + (rel + R_RAW),
        jnp.full((pad,), PAD_ROW, jnp.int32),
    ])
    tidx = jnp.concatenate([
        src * NR + rel, dst * NR + (rel + R_RAW),
        jnp.full((pad,), PAD_ROW, jnp.int32),
    ])
    gidx2 = gidx.reshape(EP // BATCH, BATCH)
    tidx2 = tidx.reshape(EP // BATCH, BATCH)
    zeros2 = jnp.zeros((ROWS_PER_SUB, EMB), f32)
    zeros1 = jnp.zeros((TBL_P,), f32)

    y1 = _dense_call(features, w1, F_IN)
    acc1, counts = _edge_pass_count(y1, gidx2, tidx2, zeros2, zeros1)
    h = _finish_relu(acc1, counts, y1, bias1, zeros1)
    y2 = _dense_call(h, w2, EMB)
    acc2 = _edge_pass_plain(y2, gidx2, tidx2, zeros2, zeros1)
    return _finish_plain(acc2, counts, y2, bias2, zeros1)


# speedup vs baseline: 1.1402x; 1.0057x over previous
import functools

import jax
import jax.numpy as jnp
from jax import lax
from jax.experimental import pallas as pl
from jax.experimental.pallas import tpu as pltpu
from jax.experimental.pallas import tpu_sc as plsc

N = 10000
F_IN = 128
EMB = 16
NCLS = 16
R_RAW = 2
NR = 2 * R_RAW + 1
E_RAW = 320000
TBL = N * NR

NC = 2
NS = 16
NT = NC * NS
LANES = 16

BATCH = 128
KB0 = 192
KB1 = 128
EP = (KB0 + KB1) * NS * BATCH

NP0, NP1 = 313, 312
SZ0, SZ1 = 5 * NP0, 5 * NP1
SZP = 1568
KC = SZP // LANES
CLOAD = 1576
SZC = 1584
KCC = SZC // LANES
ROWS_PER_SUB = TBL // NS
TBL_P = 50176
CNT_PER_SUB = TBL_P // NS
PAD_ROW = NR - 1

_mesh = plsc.VectorSubcoreMesh(
    core_axis_name="c", subcore_axis_name="s", num_cores=NC, num_subcores=NS
)


NBUF = 4
CH = 32


def _make_edge_pass(with_counts):
    out_types = [jax.ShapeDtypeStruct((NC, NS, ROWS_PER_SUB, EMB), jnp.float32)]
    scratch = (
        [pltpu.VMEM((CH, BATCH), jnp.int32) for _ in range(4)]
        + [pltpu.VMEM((BATCH, EMB), jnp.float32) for _ in range(NBUF)]
        + [pltpu.VMEM_SHARED((TBL, EMB), jnp.float32)]
        + [pltpu.SemaphoreType.DMA for _ in range(NBUF + 2)]
    )
    if with_counts:
        out_types.append(jax.ShapeDtypeStruct((NT, TBL_P), jnp.float32))
        scratch.append(pltpu.VMEM((TBL_P,), jnp.float32))

    def body(y_hbm, g_hbm, t_hbm, z_hbm, z1_hbm, *refs):
        if with_counts:
            acc_out, cnt_out = refs[0], refs[1]
            refs = refs[2:]
        else:
            acc_out = refs[0]
            refs = refs[1:]
        gi = refs[0:2]
        ti = refs[2:4]
        bufs = refs[4:4 + NBUF]
        acc_sh = refs[4 + NBUF]
        sems = refs[5 + NBUF:5 + 2 * NBUF]
        isems = refs[5 + 2 * NBUF:7 + 2 * NBUF]
        hist_v = refs[7 + 2 * NBUF] if with_counts else None

        c = lax.axis_index("c")
        s = lax.axis_index("s")
        w = c * NS + s
        pltpu.sync_copy(z_hbm, acc_sh.at[pl.ds(s * ROWS_PER_SUB, ROWS_PER_SUB)])
        if with_counts:
            pltpu.sync_copy(z1_hbm, hist_v)
        plsc.subcore_barrier()

        ones = jnp.full((LANES,), 1.0, jnp.float32)

        def pipeline(base, kb):
            nchk = kb // CH
            pltpu.async_copy(g_hbm.at[pl.ds(base, CH)], gi[0], isems[0])
            pltpu.async_copy(t_hbm.at[pl.ds(base, CH)], ti[0], isems[0])
            for ch in range(nchk):
                p = ch % 2
                if ch + 1 < nchk:
                    off = base + (ch + 1) * CH
                    pltpu.async_copy(g_hbm.at[pl.ds(off, CH)], gi[1 - p],
                                     isems[1 - p])
                    pltpu.async_copy(t_hbm.at[pl.ds(off, CH)], ti[1 - p],
                                     isems[1 - p])
                off = base + ch * CH
                pltpu.make_async_copy(
                    g_hbm.at[pl.ds(off, CH)], gi[p], isems[p]).wait()
                pltpu.make_async_copy(
                    t_hbm.at[pl.ds(off, CH)], ti[p], isems[p]).wait()

                for b in range(NBUF):
                    pltpu.async_copy(y_hbm.at[gi[p].at[b]], bufs[b], sems[b])

                @pl.loop(0, CH, step=NBUF)
                def _(j):
                    for b in range(NBUF):
                        jj = j + b
                        pltpu.make_async_copy(
                            y_hbm.at[gi[p].at[jj]], bufs[b], sems[b]).wait()
                        pltpu.sync_copy(bufs[b], acc_sh.at[ti[p].at[jj]],
                                        add=True)

                        @pl.when(jj + NBUF < CH)
                        def _():
                            pltpu.async_copy(
                                y_hbm.at[gi[p].at[jj + NBUF]], bufs[b], sems[b])

                        if with_counts:
                            for k in range(BATCH // LANES):
                                v = ti[p][jj, pl.ds(k * LANES, LANES)]
                                plsc.addupdate_scatter(hist_v, [v], ones)

        @pl.when(c == 0)
        def _():
            pipeline(s * KB0, KB0)

        @pl.when(c == 1)
        def _():
            pipeline(NS * KB0 + s * KB1, KB1)

        plsc.subcore_barrier()
        sl = pl.ds(s * ROWS_PER_SUB, ROWS_PER_SUB)
        pltpu.sync_copy(acc_sh.at[sl], acc_out.at[c, s])
        if with_counts:
            pltpu.sync_copy(hist_v, cnt_out.at[w])

    return pl.kernel(
        body,
        out_type=tuple(out_types) if with_counts else out_types[0],
        mesh=_mesh,
        scratch_types=scratch,
        compiler_params=pltpu.CompilerParams(
            use_tc_tiling_on_sc=False, needs_layout_passes=False
        ),
    )


_edge_pass_count = _make_edge_pass(True)
_edge_pass_plain = _make_edge_pass(False)


def _splat(vec, lane):
    return lax.gather(
        vec, jnp.full((LANES, 1), lane, jnp.int32),
        lax.GatherDimensionNumbers(offset_dims=(), collapsed_slice_dims=(0,),
                                   start_index_map=(0,)),
        (1,), mode=lax.GatherScatterMode.PROMISE_IN_BOUNDS)


def _make_finish(relu):
    scratch = [
        pltpu.VMEM((SZP, EMB), jnp.float32),
        pltpu.VMEM((SZP, EMB), jnp.float32),
        pltpu.VMEM((SZP, EMB), jnp.float32),
        pltpu.VMEM((8, SZC), jnp.float32),
        pltpu.VMEM((SZC,), jnp.float32),
        pltpu.VMEM((SZC, EMB), jnp.float32),
        pltpu.VMEM((320, EMB), jnp.float32),
        pltpu.VMEM((LANES,), jnp.float32),
        pltpu.SemaphoreType.DMA,
    ]

    def body(acc_hbm, cnt_hbm, y_hbm, b_hbm, z1_hbm, out_hbm,
             a0_v, a1_v, y_v, cb_v, cs_v, inv_v, h_v, b_v, sem):
        c = lax.axis_index("c")
        s = lax.axis_index("s")
        pltpu.sync_copy(b_hbm, b_v)

        def run(half, sz, np_, nbase):
            row_lo = s * ROWS_PER_SUB + half
            loads = [
                (acc_hbm.at[0, s].at[pl.ds(half, sz)], a0_v.at[pl.ds(0, sz)]),
                (acc_hbm.at[1, s].at[pl.ds(half, sz)], a1_v.at[pl.ds(0, sz)]),
                (y_hbm.at[pl.ds(row_lo, sz)], y_v.at[pl.ds(0, sz)]),
                (z1_hbm.at[pl.ds(0, SZC)], cs_v),
            ] + [(z1_hbm.at[pl.ds(0, SZC)], cb_v.at[u]) for u in range(8)]
            for src, dst in loads:
                pltpu.async_copy(src, dst, sem)
            for src, dst in loads:
                pltpu.make_async_copy(src, dst, sem).wait()
            clo = pl.multiple_of(row_lo & ~7, 8)
            dc = row_lo - clo
            for g in range(NT // 8):
                for u in range(8):
                    pltpu.async_copy(
                        cnt_hbm.at[g * 8 + u].at[pl.ds(clo, CLOAD)],
                        cb_v.at[u].at[pl.ds(0, CLOAD)], sem)
                for u in range(8):
                    pltpu.make_async_copy(
                        cnt_hbm.at[g * 8 + u].at[pl.ds(clo, CLOAD)],
                        cb_v.at[u].at[pl.ds(0, CLOAD)], sem).wait()

                @pl.loop(0, KCC)
                def _(k):
                    sl = pl.ds(k * LANES, LANES)
                    acc = cs_v[sl]
                    for u in range(8):
                        acc = acc + cb_v[u, sl]
                    cs_v[sl] = acc

            @pl.loop(0, KCC)
            def _(k):
                sl = pl.ds(k * LANES, LANES)
                cv = cs_v[sl]
                iv = jnp.where(cv > 0.0, 1.0 / cv, 0.0)
                for l in range(LANES):
                    inv_v[k * LANES + l] = _splat(iv, l)

            bias = b_v[...]

            @pl.loop(0, np_)
            def _(i):
                j = 5 * i
                msg = y_v[j + NR - 1] + bias
                for r in range(NR - 1):
                    msg = msg + (a0_v[j + r] + a1_v[j + r]) * inv_v[dc + j + r]
                if relu:
                    msg = jnp.maximum(msg, 0.0)
                h_v[i] = msg

            pltpu.sync_copy(h_v.at[pl.ds(0, np_)],
                            out_hbm.at[pl.ds(nbase, np_)])

        @pl.when(c == 0)
        def _():
            run(0, SZ0, NP0, s * 625)

        @pl.when(c == 1)
        def _():
            run(SZ0, SZ1, NP1, s * 625 + NP0)

    return pl.kernel(
        body,
        out_type=jax.ShapeDtypeStruct((N, EMB), jnp.float32),
        mesh=_mesh,
        scratch_types=scratch,
        compiler_params=pltpu.CompilerParams(
            use_tc_tiling_on_sc=False, needs_layout_passes=False
        ),
    )


_finish_relu = _make_finish(True)
_finish_plain = _make_finish(False)


def _dense_body(x_ref, w_ref, y_ref):
    x = x_ref[...]
    for r in range(NR):
        y_ref[:, r, :] = jnp.dot(x, w_ref[r], preferred_element_type=jnp.float32)


def _dense_call(x, w, k_in):
    nb = 1000
    y = pl.pallas_call(
        _dense_body,
        grid=(N // nb,),
        in_specs=[
            pl.BlockSpec((nb, k_in), lambda i: (i, 0)),
            pl.BlockSpec((NR, k_in, EMB), lambda i: (0, 0, 0)),
        ],
        out_specs=pl.BlockSpec((nb, NR, EMB), lambda i: (i, 0, 0)),
        out_shape=jax.ShapeDtypeStruct((N, NR, EMB), jnp.float32),
    )(x, w)
    return y.reshape(TBL, EMB)


def kernel(features, src, rel, dst, w1, bias1, w2, bias2):
    f32 = jnp.float32
    pad = EP - 2 * E_RAW
    gidx = jnp.concatenate([
        dst * NR + rel, src * NR + (rel + R_RAW),
        jnp.full((pad,), PAD_ROW, jnp.int32),
    ])
    tidx = jnp.concatenate([
        src * NR + rel, dst * NR + (rel + R_RAW),
        jnp.full((pad,), PAD_ROW, jnp.int32),
    ])
    gidx2 = gidx.reshape(EP // BATCH, BATCH)
    tidx2 = tidx.reshape(EP // BATCH, BATCH)
    zeros2 = jnp.zeros((ROWS_PER_SUB, EMB), f32)
    zeros1 = jnp.zeros((TBL_P,), f32)

    y1 = _dense_call(features, w1, F_IN)
    acc1, counts = _edge_pass_count(y1, gidx2, tidx2, zeros2, zeros1)
    h = _finish_relu(acc1, counts, y1, bias1, zeros1)
    y2 = _dense_call(h, w2, EMB)
    acc2 = _edge_pass_plain(y2, gidx2, tidx2, zeros2, zeros1)
    return _finish_plain(acc2, counts, y2, bias2, zeros1)
